# baseline (speedup 1.0000x reference)
import functools

import jax
import jax.numpy as jnp
from jax import lax
from jax.experimental import pallas as pl
from jax.experimental.pallas import tpu as pltpu
from jax.experimental.pallas import tpu_sc as plsc

_NUM_ENTITIES = 1000000
_NUM_RELATIONS = 1000
_D = 16
_BATCH = 16384
_MARGIN = 1.0
_EPSILON = 0.05

_NC = 2
_NS = 16
_NW = _NC * _NS
_BW = _BATCH // _NW
_CH = 128
_NCHUNK = _BW // _CH


def _make_sc_gather():
    mesh = plsc.VectorSubcoreMesh(
        core_axis_name="c", subcore_axis_name="s",
        num_cores=_NC, num_subcores=_NS)
    out_type = tuple(
        jax.ShapeDtypeStruct((_D, _BATCH), jnp.float32) for _ in range(8)
    )
    scratch = (
        [pltpu.VMEM((_BW,), jnp.int32) for _ in range(6)]
        + [pltpu.VMEM((_D, _BW), jnp.float32) for _ in range(8)]
        + [pltpu.VMEM((_D, _NUM_RELATIONS), jnp.float32) for _ in range(2)]
        + [pltpu.SemaphoreType.DMA]
    )

    @functools.partial(
        pl.kernel, mesh=mesh, out_type=out_type, scratch_types=scratch,
        compiler_params=pltpu.CompilerParams(
            use_tc_tiling_on_sc=False, needs_layout_passes=False),
    )
    def sc_gather(*refs):
        ins = refs[:9]
        outs = refs[9:17]
        scr = refs[17:]
        idx_hbm = ins[:6]
        ent_flat = ins[6]
        rel_hbm, nrm_hbm = ins[7], ins[8]
        ent = [ent_flat.at[pl.ds(d * _NUM_ENTITIES, _NUM_ENTITIES)]
               for d in range(_D)]
        idx_v = scr[:6]
        stag = scr[6:14]
        rel_v, nrm_v = scr[14], scr[15]
        sem = scr[16]

        wid = lax.axis_index("s") * _NC + lax.axis_index("c")
        base = wid * _BW

        for src, dst in zip(idx_hbm, idx_v):
            pltpu.sync_copy(src.at[pl.ds(base, _BW)], dst)
        pltpu.sync_copy(rel_hbm, rel_v)
        pltpu.sync_copy(nrm_hbm, nrm_v)

        iph, ipr, ipt, inh, inr, intl = idx_v
        ent_jobs = (
            (iph, stag[0]), (ipt, stag[2]), (inh, stag[4]), (intl, stag[6]),
        )
        descs = []
        for idxb, sg in ent_jobs:
            for c in range(_NCHUNK):
                isl = idxb.at[pl.ds(c * _CH, _CH)]
                for d in range(_D):
                    descs.append(pltpu.async_copy(
                        ent[d].at[isl],
                        sg.at[d, pl.ds(c * _CH, _CH)], sem))

        rel_jobs = (
            (ipr, rel_v, stag[1]), (ipr, nrm_v, stag[3]),
            (inr, rel_v, stag[5]), (inr, nrm_v, stag[7]),
        )
        dvecs = [jnp.full((16,), d, jnp.int32) for d in range(_D)]

        def body(g, _):
            for idxb, tab, sg in rel_jobs:
                idx16 = idxb[pl.ds(g * 16, 16)]
                for d in range(_D):
                    vals = plsc.load_gather(tab, [dvecs[d], idx16])
                    sg[d, pl.ds(g * 16, 16)] = vals
            return _

        lax.fori_loop(0, _BW // 16, body, 0)

        for dsc in descs:
            dsc.wait()

        order = (stag[0], stag[1], stag[2], stag[3],
                 stag[4], stag[5], stag[6], stag[7])
        for sg, out in zip(order, outs):
            pltpu.sync_copy(sg, out.at[:, pl.ds(base, _BW)])

    return sc_gather


_sc_gather_cache = []


def _sc_gather(*args):
    if not _sc_gather_cache:
        _sc_gather_cache.append(_make_sc_gather())
    return _sc_gather_cache[0](*args)


_CB = 65536
_GRID_B = (_NUM_ENTITIES + _CB - 1) // _CB


def _constraints_body(ent_ref, nrm_ref, prj_ref, out_ref):
    i = pl.program_id(0)
    x = ent_ref[...]
    sq = jnp.sum(x * x, axis=0, keepdims=True)
    col = i * _CB + lax.broadcasted_iota(jnp.int32, (1, _CB), 1)
    contrib = jnp.where(col < _NUM_ENTITIES,
                        jnp.abs(sq - float(_NUM_ENTITIES)), 0.0)
    part = jnp.sum(contrib)

    @pl.when(i == 0)
    def _():
        n = nrm_ref[...]
        dpr = prj_ref[...]
        nn = jnp.sum(n * n, axis=0, keepdims=True)
        nd = jnp.sum(n * dpr, axis=0, keepdims=True)
        dd = jnp.sum(dpr * dpr, axis=0, keepdims=True)
        orth = jnp.sum(
            jnp.abs(nd * nd / (nn * dd) - float(_NUM_RELATIONS) * _EPSILON))
        out_ref[...] = orth.reshape(1, 1)

    out_ref[...] += part.reshape(1, 1)


def _constraints(entT, nrmT, prjT):
    return pl.pallas_call(
        _constraints_body,
        grid=(_GRID_B,),
        in_specs=[
            pl.BlockSpec((_D, _CB), lambda i: (0, i)),
            pl.BlockSpec((_D, _NUM_RELATIONS), lambda i: (0, 0)),
            pl.BlockSpec((_D, _NUM_RELATIONS), lambda i: (0, 0)),
        ],
        out_specs=pl.BlockSpec((1, 1), lambda i: (0, 0)),
        out_shape=jax.ShapeDtypeStruct((1, 1), jnp.float32),
    )(entT, nrmT, prjT)


def _margin_body(ph, pr, pt, pn, nh, nr, nt, nn, out_ref):
    def score(h, r, t, n):
        d = h[...] - t[...]
        nv = n[...]
        ndot = jnp.sum(nv * d, axis=0, keepdims=True)
        nsq = jnp.sum(nv * nv, axis=0, keepdims=True)
        s = d + r[...] - (ndot / nsq) * nv
        return jnp.sqrt(jnp.sum(s * s, axis=0, keepdims=True))

    sp = score(ph, pr, pt, pn)
    sn = score(nh, nr, nt, nn)
    out_ref[...] = jnp.sum(
        jnp.maximum(sp - sn + _MARGIN, 0.0)).reshape(1, 1)


def _margin(*gatheredT):
    return pl.pallas_call(
        _margin_body,
        out_shape=jax.ShapeDtypeStruct((1, 1), jnp.float32),
    )(*gatheredT)


def kernel(pos_heads, pos_rels, pos_tails, neg_heads, neg_rels, neg_tails,
           entity_emb, relation_emb, normal_emb, proj_rel_emb, w_soft):
    ph = pos_heads.astype(jnp.int32)
    pr = pos_rels.astype(jnp.int32)
    pt = pos_tails.astype(jnp.int32)
    nh = neg_heads.astype(jnp.int32)
    nr = neg_rels.astype(jnp.int32)
    nt = neg_tails.astype(jnp.int32)

    entT = entity_emb.T
    relT = relation_emb.T
    nrmT = normal_emb.T
    prjT = proj_rel_emb.T

    ent_flat16 = entT.reshape(-1)

    gathered = _sc_gather(ph, pr, pt, nh, nr, nt,
                          ent_flat16, relT, nrmT)

    ent_orth = _constraints(entT, nrmT, prjT)
    margin = _margin(*gathered)

    return margin[0, 0] + w_soft[0] * ent_orth[0, 0]


# speedup vs baseline: 9.2714x; 9.2714x over previous
import functools

import jax
import jax.numpy as jnp
from jax import lax
from jax.experimental import pallas as pl
from jax.experimental.pallas import tpu as pltpu
from jax.experimental.pallas import tpu_sc as plsc

_NUM_ENTITIES = 1000000
_NUM_RELATIONS = 1000
_D = 16
_BATCH = 16384
_MARGIN = 1.0
_EPSILON = 0.05

_NC = 2
_NS = 16
_NW = _NC * _NS
_BW = _BATCH // _NW
_CH = 128
_NCHUNK = _BW // _CH


def _make_sc_gather():
    mesh = plsc.VectorSubcoreMesh(
        core_axis_name="c", subcore_axis_name="s",
        num_cores=_NC, num_subcores=_NS)
    out_type = tuple(
        jax.ShapeDtypeStruct((_D, _BATCH), jnp.float32) for _ in range(8)
    )
    scratch = (
        [pltpu.VMEM((_BW,), jnp.int32) for _ in range(6)]
        + [pltpu.VMEM((_D, _BW), jnp.float32) for _ in range(8)]
        + [pltpu.VMEM((_D, _NUM_RELATIONS), jnp.float32) for _ in range(2)]
        + [pltpu.SemaphoreType.DMA]
    )

    @functools.partial(
        pl.kernel, mesh=mesh, out_type=out_type, scratch_types=scratch,
        compiler_params=pltpu.CompilerParams(
            use_tc_tiling_on_sc=False, needs_layout_passes=False),
    )
    def sc_gather(*refs):
        ins = refs[:24]
        outs = refs[24:32]
        scr = refs[32:]
        idx_hbm = ins[:6]
        ent = ins[6:22]
        rel_hbm, nrm_hbm = ins[22], ins[23]
        idx_v = scr[:6]
        stag = scr[6:14]
        rel_v, nrm_v = scr[14], scr[15]
        sem = scr[16]

        wid = lax.axis_index("s") * _NC + lax.axis_index("c")
        base = wid * _BW

        for src, dst in zip(idx_hbm, idx_v):
            pltpu.sync_copy(src.at[pl.ds(base, _BW)], dst)
        pltpu.sync_copy(rel_hbm, rel_v)
        pltpu.sync_copy(nrm_hbm, nrm_v)

        iph, ipr, ipt, inh, inr, intl = idx_v
        ent_jobs = (
            (iph, stag[0]), (ipt, stag[2]), (inh, stag[4]), (intl, stag[6]),
        )
        descs = []
        for idxb, sg in ent_jobs:
            for c in range(_NCHUNK):
                isl = idxb.at[pl.ds(c * _CH, _CH)]
                for d in range(_D):
                    descs.append(pltpu.async_copy(
                        ent[d].at[isl],
                        sg.at[d, pl.ds(c * _CH, _CH)], sem))

        rel_jobs = (
            (ipr, rel_v, stag[1]), (ipr, nrm_v, stag[3]),
            (inr, rel_v, stag[5]), (inr, nrm_v, stag[7]),
        )
        dvecs = [jnp.full((16,), d, jnp.int32) for d in range(_D)]

        def body(g, _):
            for idxb, tab, sg in rel_jobs:
                idx16 = idxb[pl.ds(g * 16, 16)]
                for d in range(_D):
                    vals = plsc.load_gather(tab, [dvecs[d], idx16])
                    sg[d, pl.ds(g * 16, 16)] = vals
            return _

        lax.fori_loop(0, _BW // 16, body, 0)

        for dsc in descs:
            dsc.wait()

        order = (stag[0], stag[1], stag[2], stag[3],
                 stag[4], stag[5], stag[6], stag[7])
        for sg, out in zip(order, outs):
            pltpu.sync_copy(sg, out.at[:, pl.ds(base, _BW)])

    return sc_gather


_sc_gather_cache = []


def _sc_gather(*args):
    if not _sc_gather_cache:
        _sc_gather_cache.append(_make_sc_gather())
    return _sc_gather_cache[0](*args)


_CB = 65536
_GRID_B = (_NUM_ENTITIES + _CB - 1) // _CB


def _constraints_body(ent_ref, nrm_ref, prj_ref, out_ref, *row_refs):
    i = pl.program_id(0)
    x = ent_ref[...]
    for d in range(_D):
        row_refs[d][...] = x[d, :]
    sq = jnp.sum(x * x, axis=0, keepdims=True)
    col = i * _CB + lax.broadcasted_iota(jnp.int32, (1, _CB), 1)
    contrib = jnp.where(col < _NUM_ENTITIES,
                        jnp.abs(sq - float(_NUM_ENTITIES)), 0.0)
    part = jnp.sum(contrib)

    @pl.when(i == 0)
    def _():
        n = nrm_ref[...]
        dpr = prj_ref[...]
        nn = jnp.sum(n * n, axis=0, keepdims=True)
        nd = jnp.sum(n * dpr, axis=0, keepdims=True)
        dd = jnp.sum(dpr * dpr, axis=0, keepdims=True)
        orth = jnp.sum(
            jnp.abs(nd * nd / (nn * dd) - float(_NUM_RELATIONS) * _EPSILON))
        out_ref[...] = orth.reshape(1, 1)

    out_ref[...] += part.reshape(1, 1)


def _constraints(entT, nrmT, prjT):
    return pl.pallas_call(
        _constraints_body,
        grid=(_GRID_B,),
        in_specs=[
            pl.BlockSpec((_D, _CB), lambda i: (0, i)),
            pl.BlockSpec((_D, _NUM_RELATIONS), lambda i: (0, 0)),
            pl.BlockSpec((_D, _NUM_RELATIONS), lambda i: (0, 0)),
        ],
        out_specs=[pl.BlockSpec((1, 1), lambda i: (0, 0))]
        + [pl.BlockSpec((_CB,), lambda i: (i,)) for _ in range(_D)],
        out_shape=[jax.ShapeDtypeStruct((1, 1), jnp.float32)]
        + [jax.ShapeDtypeStruct((_NUM_ENTITIES,), jnp.float32)
           for _ in range(_D)],
    )(entT, nrmT, prjT)


def _margin_body(ph, pr, pt, pn, nh, nr, nt, nn, out_ref):
    def score(h, r, t, n):
        d = h[...] - t[...]
        nv = n[...]
        ndot = jnp.sum(nv * d, axis=0, keepdims=True)
        nsq = jnp.sum(nv * nv, axis=0, keepdims=True)
        s = d + r[...] - (ndot / nsq) * nv
        return jnp.sqrt(jnp.sum(s * s, axis=0, keepdims=True))

    sp = score(ph, pr, pt, pn)
    sn = score(nh, nr, nt, nn)
    out_ref[...] = jnp.sum(
        jnp.maximum(sp - sn + _MARGIN, 0.0)).reshape(1, 1)


def _margin(*gatheredT):
    return pl.pallas_call(
        _margin_body,
        out_shape=jax.ShapeDtypeStruct((1, 1), jnp.float32),
    )(*gatheredT)


def kernel(pos_heads, pos_rels, pos_tails, neg_heads, neg_rels, neg_tails,
           entity_emb, relation_emb, normal_emb, proj_rel_emb, w_soft):
    ph = pos_heads.astype(jnp.int32)
    pr = pos_rels.astype(jnp.int32)
    pt = pos_tails.astype(jnp.int32)
    nh = neg_heads.astype(jnp.int32)
    nr = neg_rels.astype(jnp.int32)
    nt = neg_tails.astype(jnp.int32)

    entT = entity_emb.T
    relT = relation_emb.T
    nrmT = normal_emb.T
    prjT = proj_rel_emb.T

    ent_orth, *ent_rows = _constraints(entT, nrmT, prjT)

    gathered = _sc_gather(ph, pr, pt, nh, nr, nt,
                          *ent_rows, relT, nrmT)

    margin = _margin(*gathered)

    return margin[0, 0] + w_soft[0] * ent_orth[0, 0]


# speedup vs baseline: 10.2935x; 1.1102x over previous
import functools

import jax
import jax.numpy as jnp
from jax import lax
from jax.experimental import pallas as pl
from jax.experimental.pallas import tpu as pltpu
from jax.experimental.pallas import tpu_sc as plsc

_NUM_ENTITIES = 1000000
_NUM_RELATIONS = 1000
_D = 16
_BATCH = 16384
_MARGIN = 1.0
_EPSILON = 0.05

_NC = 2
_NS = 16
_NW = _NC * _NS
_BW = _BATCH // _NW
_CH = 128
_NCHUNK = _BW // _CH


def _make_sc_gather():
    mesh = plsc.VectorSubcoreMesh(
        core_axis_name="c", subcore_axis_name="s",
        num_cores=_NC, num_subcores=_NS)
    out_type = (jax.ShapeDtypeStruct((_NW, 16), jnp.float32),)
    scratch = (
        [pltpu.VMEM((_BW,), jnp.int32) for _ in range(6)]
        + [pltpu.VMEM((_D, _BW), jnp.float32) for _ in range(8)]
        + [pltpu.VMEM((_D, _NUM_RELATIONS), jnp.float32) for _ in range(2)]
        + [pltpu.VMEM((16,), jnp.float32)]
        + [pltpu.SemaphoreType.DMA]
    )

    @functools.partial(
        pl.kernel, mesh=mesh, out_type=out_type, scratch_types=scratch,
        compiler_params=pltpu.CompilerParams(
            use_tc_tiling_on_sc=False, needs_layout_passes=False),
    )
    def sc_gather(*refs):
        ins = refs[:24]
        out_m = refs[24]
        scr = refs[25:]
        idx_hbm = ins[:6]
        ent = ins[6:22]
        rel_hbm, nrm_hbm = ins[22], ins[23]
        idx_v = scr[:6]
        stag = scr[6:14]
        rel_v, nrm_v = scr[14], scr[15]
        acc_v = scr[16]
        sem = scr[17]

        wid = lax.axis_index("s") * _NC + lax.axis_index("c")
        base = wid * _BW

        for src, dst in zip(idx_hbm, idx_v):
            pltpu.sync_copy(src.at[pl.ds(base, _BW)], dst)
        pltpu.sync_copy(rel_hbm, rel_v)
        pltpu.sync_copy(nrm_hbm, nrm_v)

        iph, ipr, ipt, inh, inr, intl = idx_v
        ent_jobs = (
            (iph, stag[0]), (ipt, stag[2]), (inh, stag[4]), (intl, stag[6]),
        )
        descs = []
        for idxb, sg in ent_jobs:
            for c in range(_NCHUNK):
                isl = idxb.at[pl.ds(c * _CH, _CH)]
                for d in range(_D):
                    descs.append(pltpu.async_copy(
                        ent[d].at[isl],
                        sg.at[d, pl.ds(c * _CH, _CH)], sem))

        rel_jobs = (
            (ipr, rel_v, stag[1]), (ipr, nrm_v, stag[3]),
            (inr, rel_v, stag[5]), (inr, nrm_v, stag[7]),
        )
        dvecs = [jnp.full((16,), d, jnp.int32) for d in range(_D)]

        def body(g, _):
            for idxb, tab, sg in rel_jobs:
                idx16 = idxb[pl.ds(g * 16, 16)]
                for d in range(_D):
                    vals = plsc.load_gather(tab, [dvecs[d], idx16])
                    sg[d, pl.ds(g * 16, 16)] = vals
            return _

        lax.fori_loop(0, _BW // 16, body, 0)

        for dsc in descs:
            dsc.wait()

        def nsqrt(x):
            xc = jnp.maximum(x, 1e-30)
            i = plsc.bitcast(xc, jnp.int32)
            y = plsc.bitcast(jnp.int32(0x5F3759DF) - (i >> 1), jnp.float32)
            for _ in range(3):
                y = y * (1.5 - 0.5 * xc * y * y)
            return x * y

        zero16 = jnp.zeros((16,), jnp.float32)

        def margin_body(g, acc):
            sl = pl.ds(g * 16, 16)
            nd_p = zero16
            nn_p = zero16
            nd_n = zero16
            nn_n = zero16
            for d in range(_D):
                dp = stag[0][d, sl] - stag[2][d, sl]
                pnv = stag[3][d, sl]
                nd_p = nd_p + pnv * dp
                nn_p = nn_p + pnv * pnv
                dn = stag[4][d, sl] - stag[6][d, sl]
                nnv = stag[7][d, sl]
                nd_n = nd_n + nnv * dn
                nn_n = nn_n + nnv * nnv
            cp = nd_p / nn_p
            cn = nd_n / nn_n
            sq_p = zero16
            sq_n = zero16
            for d in range(_D):
                sp = (stag[0][d, sl] - stag[2][d, sl]) + stag[1][d, sl] \
                    - cp * stag[3][d, sl]
                sq_p = sq_p + sp * sp
                sn = (stag[4][d, sl] - stag[6][d, sl]) + stag[5][d, sl] \
                    - cn * stag[7][d, sl]
                sq_n = sq_n + sn * sn
            return acc + jnp.maximum(nsqrt(sq_p) - nsqrt(sq_n) + _MARGIN, 0.0)

        acc = lax.fori_loop(0, _BW // 16, margin_body, zero16)
        acc_v[...] = acc
        pltpu.sync_copy(acc_v, out_m.at[wid])

    return sc_gather


_sc_gather_cache = []


def _sc_gather(*args):
    if not _sc_gather_cache:
        _sc_gather_cache.append(_make_sc_gather())
    return _sc_gather_cache[0](*args)


_CB = 65536
_GRID_B = (_NUM_ENTITIES + _CB - 1) // _CB


def _constraints_body(ent_ref, nrm_ref, prj_ref, out_ref, *row_refs):
    i = pl.program_id(0)
    x = ent_ref[...]
    for d in range(_D):
        row_refs[d][...] = x[d, :]
    sq = jnp.sum(x * x, axis=0, keepdims=True)
    col = i * _CB + lax.broadcasted_iota(jnp.int32, (1, _CB), 1)
    contrib = jnp.where(col < _NUM_ENTITIES,
                        jnp.abs(sq - float(_NUM_ENTITIES)), 0.0)
    part = jnp.sum(contrib)

    @pl.when(i == 0)
    def _():
        n = nrm_ref[...]
        dpr = prj_ref[...]
        nn = jnp.sum(n * n, axis=0, keepdims=True)
        nd = jnp.sum(n * dpr, axis=0, keepdims=True)
        dd = jnp.sum(dpr * dpr, axis=0, keepdims=True)
        orth = jnp.sum(
            jnp.abs(nd * nd / (nn * dd) - float(_NUM_RELATIONS) * _EPSILON))
        out_ref[...] = orth.reshape(1, 1)

    out_ref[...] += part.reshape(1, 1)


def _constraints(entT, nrmT, prjT):
    return pl.pallas_call(
        _constraints_body,
        grid=(_GRID_B,),
        in_specs=[
            pl.BlockSpec((_D, _CB), lambda i: (0, i)),
            pl.BlockSpec((_D, _NUM_RELATIONS), lambda i: (0, 0)),
            pl.BlockSpec((_D, _NUM_RELATIONS), lambda i: (0, 0)),
        ],
        out_specs=[pl.BlockSpec((1, 1), lambda i: (0, 0))]
        + [pl.BlockSpec((_CB,), lambda i: (i,)) for _ in range(_D)],
        out_shape=[jax.ShapeDtypeStruct((1, 1), jnp.float32)]
        + [jax.ShapeDtypeStruct((_NUM_ENTITIES,), jnp.float32)
           for _ in range(_D)],
    )(entT, nrmT, prjT)


def _margin_body(ph, pr, pt, pn, nh, nr, nt, nn, out_ref):
    def score(h, r, t, n):
        d = h[...] - t[...]
        nv = n[...]
        ndot = jnp.sum(nv * d, axis=0, keepdims=True)
        nsq = jnp.sum(nv * nv, axis=0, keepdims=True)
        s = d + r[...] - (ndot / nsq) * nv
        return jnp.sqrt(jnp.sum(s * s, axis=0, keepdims=True))

    sp = score(ph, pr, pt, pn)
    sn = score(nh, nr, nt, nn)
    out_ref[...] = jnp.sum(
        jnp.maximum(sp - sn + _MARGIN, 0.0)).reshape(1, 1)


def _margin(*gatheredT):
    return pl.pallas_call(
        _margin_body,
        out_shape=jax.ShapeDtypeStruct((1, 1), jnp.float32),
    )(*gatheredT)


def kernel(pos_heads, pos_rels, pos_tails, neg_heads, neg_rels, neg_tails,
           entity_emb, relation_emb, normal_emb, proj_rel_emb, w_soft):
    ph = pos_heads.astype(jnp.int32)
    pr = pos_rels.astype(jnp.int32)
    pt = pos_tails.astype(jnp.int32)
    nh = neg_heads.astype(jnp.int32)
    nr = neg_rels.astype(jnp.int32)
    nt = neg_tails.astype(jnp.int32)

    entT = entity_emb.T
    relT = relation_emb.T
    nrmT = normal_emb.T
    prjT = proj_rel_emb.T

    ent_orth, *ent_rows = _constraints(entT, nrmT, prjT)

    (margin_parts,) = _sc_gather(ph, pr, pt, nh, nr, nt,
                                 *ent_rows, relT, nrmT)

    return jnp.sum(margin_parts) + w_soft[0] * ent_orth[0, 0]
